# baseline (speedup 1.0000x reference)
import functools

import jax
import jax.numpy as jnp
from jax import lax
from jax.experimental import pallas as pl
from jax.experimental.pallas import tpu as pltpu
from jax.experimental.pallas import tpu_sc as plsc

EMBED_DIM = 64
HALF = EMBED_DIM // 2
BATCH = 16384
MODE_PAD = 8
TABLE_ROWS = 16 * MODE_PAD

NC = 2
NS = 16
L = 16
NW = NC * NS
BPW = BATCH // NW
GCH = 128
NG = BPW // GCH
DB = EMBED_DIM // 8


def _table_body(miss_ref, mode_ref, w_ref, b_ref, out_ref):
    w1 = w_ref[:, :HALF]
    w2 = w_ref[:, HALF:]
    dn = (((1,), (1,)), ((), ()))
    miss_proj = lax.dot_general(miss_ref[...], w1, dn,
                                preferred_element_type=jnp.float32)
    mode_proj5 = lax.dot_general(mode_ref[...], w2, dn,
                                 preferred_element_type=jnp.float32)
    mode_proj = jnp.concatenate(
        [mode_proj5, jnp.zeros((MODE_PAD - 5, EMBED_DIM), jnp.float32)], 0)
    t3 = miss_proj[:, None, :] + mode_proj[None, :, :] + b_ref[...][None]
    out_ref[...] = t3.reshape(TABLE_ROWS, EMBED_DIM).T


def _build_table_t(missing_table, mode_table, W, b):
    return pl.pallas_call(
        _table_body,
        out_shape=jax.ShapeDtypeStruct((EMBED_DIM, TABLE_ROWS), jnp.float32),
    )(missing_table, mode_table, W, b.reshape(1, EMBED_DIM))


@functools.cache
def _make_sc_gather():
    mesh = plsc.VectorSubcoreMesh(core_axis_name="c", subcore_axis_name="s")

    @functools.partial(
        pl.kernel,
        mesh=mesh,
        compiler_params=pltpu.CompilerParams(use_tc_tiling_on_sc=False,
                                             needs_layout_passes=False),
        out_type=jax.ShapeDtypeStruct((DB, BATCH // GCH, 8, GCH), jnp.float32),
        scratch_types=[
            pltpu.VMEM((NG, 4, GCH), jnp.int32),
            pltpu.VMEM((BPW,), jnp.int32),
            pltpu.VMEM((NG, GCH), jnp.int32),
            pltpu.VMEM((EMBED_DIM, TABLE_ROWS), jnp.float32),
            pltpu.VMEM((2, DB, 8, GCH), jnp.float32),
            pltpu.SemaphoreType.DMA,
            pltpu.SemaphoreType.DMA,
        ],
    )
    def _sc_gather(maskb_hbm, mode_hbm, tablet_hbm, out_hbm,
                   mask_v, mode_v, idx_v, tab_v, bands_v, ssem, osem):
        sid = lax.axis_index("s")
        wid = sid * NC + lax.axis_index("c")

        stage = [
            pltpu.async_copy(tablet_hbm, tab_v, ssem),
            pltpu.async_copy(maskb_hbm.at[pl.ds(wid * NG, NG)], mask_v, ssem),
            pltpu.async_copy(mode_hbm.at[pl.ds(wid * BPW, BPW)], mode_v, ssem),
        ]
        for c in stage:
            c.wait()

        for g in range(NG):
            for i in range(GCH // L):
                off = i * L
                m0 = mask_v[g, 0, pl.ds(off, L)]
                m1 = mask_v[g, 1, pl.ds(off, L)]
                m2 = mask_v[g, 2, pl.ds(off, L)]
                m3 = mask_v[g, 3, pl.ds(off, L)]
                md = mode_v[pl.ds(g * GCH + off, L)]
                idx_v[g, pl.ds(off, L)] = (
                    m0 * 64 + m1 * 32 + m2 * 16 + m3 * 8 + md)

        out_copies = []
        for g in range(NG):
            pg = g % 2
            if g >= 2:
                for c in out_copies[(g - 2) * DB:(g - 1) * DB]:
                    c.wait()
            blk = wid * NG + g
            for j in range(GCH // L):
                idx16 = idx_v[g, pl.ds(j * L, L)]
                for a in range(DB):
                    for r in range(8):
                        dsplat = jnp.full((L,), a * 8 + r, jnp.int32)
                        vals = plsc.load_gather(tab_v, [dsplat, idx16])
                        bands_v[pg, a, r, pl.ds(j * L, L)] = vals
            for a in range(DB):
                out_copies.append(pltpu.async_copy(
                    bands_v.at[pg, a], out_hbm.at[a, blk], osem))
        for c in out_copies[(NG - 2) * DB:]:
            c.wait()

    return _sc_gather


def kernel(missing_mask, mode_id, missing_table, mode_table, W, b):
    tablet = _build_table_t(missing_table, mode_table, W, b)
    mask32 = missing_mask.astype(jnp.int32)
    maskb = jnp.transpose(mask32.reshape(BATCH // GCH, GCH, 4), (0, 2, 1))
    mode32 = mode_id.astype(jnp.int32)
    out4 = _make_sc_gather()(maskb, mode32, tablet)
    return jnp.transpose(out4, (1, 3, 0, 2)).reshape(BATCH, EMBED_DIM)


# speedup vs baseline: 1.4142x; 1.4142x over previous
import functools

import jax
import jax.numpy as jnp
from jax import lax
from jax.experimental import pallas as pl
from jax.experimental.pallas import tpu as pltpu
from jax.experimental.pallas import tpu_sc as plsc

EMBED_DIM = 64
HALF = EMBED_DIM // 2
BATCH = 16384
MODE_PAD = 8
TABLE_ROWS = 16 * MODE_PAD

NC = 2
NS = 16
L = 16
NW = NC * NS
BPW = BATCH // NW
GCH = 128
NG = BPW // GCH


def _table_body(miss_ref, mode_ref, w_ref, b_ref, out_ref):
    w1 = w_ref[:, :HALF]
    w2 = w_ref[:, HALF:]
    dn = (((1,), (1,)), ((), ()))
    miss_proj = lax.dot_general(miss_ref[...], w1, dn,
                                preferred_element_type=jnp.float32)
    mode_proj5 = lax.dot_general(mode_ref[...], w2, dn,
                                 preferred_element_type=jnp.float32)
    mode_proj = jnp.concatenate(
        [mode_proj5, jnp.zeros((MODE_PAD - 5, EMBED_DIM), jnp.float32)], 0)
    t3 = miss_proj[:, None, :] + mode_proj[None, :, :] + b_ref[...][None]
    t64 = t3.reshape(TABLE_ROWS, EMBED_DIM)
    out_ref[...] = jnp.concatenate(
        [t64, jnp.zeros((TABLE_ROWS, 128 - EMBED_DIM), jnp.float32)], 1)


def _build_table(missing_table, mode_table, W, b):
    return pl.pallas_call(
        _table_body,
        out_shape=jax.ShapeDtypeStruct((TABLE_ROWS, 128), jnp.float32),
    )(missing_table, mode_table, W, b.reshape(1, EMBED_DIM))


@functools.cache
def _make_sc_gather():
    mesh = plsc.VectorSubcoreMesh(core_axis_name="c", subcore_axis_name="s")

    @functools.partial(
        pl.kernel,
        mesh=mesh,
        compiler_params=pltpu.CompilerParams(use_tc_tiling_on_sc=False),
        out_type=jax.ShapeDtypeStruct((BATCH, 128), jnp.float32),
        scratch_types=[
            pltpu.VMEM((NG, 4, GCH), jnp.int32),
            pltpu.VMEM((BPW,), jnp.int32),
            pltpu.VMEM((NG, GCH), jnp.int32),
            pltpu.VMEM((BPW, EMBED_DIM), jnp.float32),
            pltpu.VMEM_SHARED((TABLE_ROWS, EMBED_DIM), jnp.float32),
            pltpu.SemaphoreType.DMA,
            pltpu.SemaphoreType.DMA,
        ],
    )
    def _sc_gather(maskb_hbm, mode_hbm, table_hbm, out_hbm,
                   mask_v, mode_v, idx_v, rows_v, table_s, ssem, gsem):
        sid = lax.axis_index("s")
        wid = sid * NC + lax.axis_index("c")
        base = wid * BPW

        @pl.when(sid == 0)
        def _():
            pltpu.sync_copy(table_hbm.at[:, pl.ds(0, EMBED_DIM)], table_s)

        stage = [
            pltpu.async_copy(maskb_hbm.at[pl.ds(wid * NG, NG)], mask_v, ssem),
            pltpu.async_copy(mode_hbm.at[pl.ds(base, BPW)], mode_v, ssem),
        ]
        for c in stage:
            c.wait()

        copies = []
        for g in range(NG):
            for i in range(GCH // L):
                off = i * L
                m0 = mask_v[g, 0, pl.ds(off, L)]
                m1 = mask_v[g, 1, pl.ds(off, L)]
                m2 = mask_v[g, 2, pl.ds(off, L)]
                m3 = mask_v[g, 3, pl.ds(off, L)]
                md = mode_v[pl.ds(g * GCH + off, L)]
                idx_v[g, pl.ds(off, L)] = (
                    m0 * 64 + m1 * 32 + m2 * 16 + m3 * 8 + md)
            if g == 0:
                plsc.subcore_barrier()
            copies.append(pltpu.async_copy(
                table_s.at[idx_v.at[g]], rows_v.at[pl.ds(g * GCH, GCH)],
                gsem))
        out_copies = []
        for g in range(NG):
            copies[g].wait()
            out_copies.append(pltpu.async_copy(
                rows_v.at[pl.ds(g * GCH, GCH)],
                out_hbm.at[pl.ds(base + g * GCH, GCH), pl.ds(0, EMBED_DIM)],
                ssem))
        for c in out_copies:
            c.wait()

    return _sc_gather


def kernel(missing_mask, mode_id, missing_table, mode_table, W, b):
    table = _build_table(missing_table, mode_table, W, b)
    mask32 = missing_mask.astype(jnp.int32)
    maskb = jnp.transpose(mask32.reshape(BATCH // GCH, GCH, 4), (0, 2, 1))
    mode32 = mode_id.astype(jnp.int32)
    out2 = _make_sc_gather()(maskb, mode32, table)
    return out2[:, :EMBED_DIM]
